# baseline (speedup 1.0000x reference)
import jax
import jax.numpy as jnp
import numpy as np
from jax.experimental import pallas as pl
from jax.experimental.pallas import tpu as pltpu

_NCODONS = 2
_N_STATES = 24
_N_PARAMS = 10


def _static_structure(nCodons=_NCODONS):
    offset = 8 + 3 * nCodons
    idx = [[0, 0], [0, 1], [1, 2], [2, 3]]
    idx += [[3 + i * 3, 4 + i * 3] for i in range(nCodons)]
    idx += [[4 + i * 3, 5 + i * 3] for i in range(nCodons)]
    idx += [[5 + i * 3, 6 + i * 3] for i in range(nCodons)]
    idx += [[3 + i * 3, offset + i * 3] for i in range(nCodons + 1)]
    idx += [[3 + nCodons * 3, 4 + nCodons * 3]]
    idx += [[offset + i * 3, offset + 1 + i * 3] for i in range(nCodons + 1)]
    idx += [[offset + 1 + i * 3, offset + 2 + i * 3] for i in range(nCodons + 1)]
    idx += [[offset + 2 + i * 3, 4 + i * 3] for i in range(nCodons + 1)]
    idx += [[offset + 2 + i * 3, offset + i * 3] for i in range(nCodons + 1)]
    i_del = [3 + i * 3 for i in range(nCodons) for j in range(nCodons - i)]
    j_del = [4 + j * 3 for i in range(1, nCodons + 1) for j in range(i, nCodons + 1)]
    idx += [[i, j] for i, j in zip(i_del, j_del)]
    idx += [[4 + nCodons * 3, 5 + nCodons * 3]]
    idx += [[5 + nCodons * 3, 6 + nCodons * 3]]
    idx += [[6 + nCodons * 3, 7 + nCodons * 3]]
    idx += [[7 + nCodons * 3, 7 + nCodons * 3]]
    idx += [[7 + nCodons * 3, 8 + nCodons * 3 + (nCodons + 1) * 3]]
    idx += [[8 + nCodons * 3 + (nCodons + 1) * 3,
             8 + nCodons * 3 + (nCodons + 1) * 3]]
    idx = np.array(idx, dtype=np.int32)

    nc = nCodons
    a, b, p, e = [], [], [], []

    def add(ai, bi, pi, ei):
        a.append(ai); b.append(bi); p.append(pi); e.append(ei)

    add(1.0, -1.0, 0, 1)
    add(0.0, 1.0, 0, 1)
    for _ in range(2):
        add(1.0, 0.0, 0, 1)
    k = 1
    for i in range(nc):
        add(0.0, 1.0, k + i, 1)
    k += nc
    for _ in range(2 * nc):
        add(1.0, 0.0, 0, 1)
    for i in range(nc + 1):
        add(0.0, 1.0, k + i, 1)
    k += nc + 1
    add(1.0, -1.0, k - 1, 1)
    for _ in range(2 * (nc + 1)):
        add(1.0, 0.0, 0, 1)
    for i in range(nc + 1):
        add(0.0, 1.0, k + i, 1)
    for i in range(nc + 1):
        add(1.0, -1.0, k + i, 1)
    k += nc + 1
    for i, j in zip(i_del, j_del):
        add(1.0, -1.0, k, 1 + int((j - i) / 3))
    k += 1
    for _ in range(6):
        add(1.0, 0.0, 0, 1)

    assert len(a) == len(idx)
    return (idx, np.asarray(a, np.float32), np.asarray(b, np.float32),
            np.asarray(p, np.int32), np.asarray(e, np.int32))


_IDX, _COEF_A, _COEF_B, _PAR, _EXP = _static_structure()
_NK = len(_IDX)


_W = 384
_NPH = 3

_LIVE_TILES = [(q, p) for q in range(_NPH) for p in range(_NPH)
               if not (q == 0 and p == 2) and not (q == 2 and p == 0)]


def _amat(w_ref):
        ws = [w_ref[0, i] for i in range(_N_PARAMS)]
        ri = jax.lax.broadcasted_iota(jnp.int32, (_N_STATES, _N_STATES), 0)
        ci = jax.lax.broadcasted_iota(jnp.int32, (_N_STATES, _N_STATES), 1)
        logits = jnp.zeros((_N_STATES, _N_STATES), jnp.float32)
        maskf = jnp.zeros((_N_STATES, _N_STATES), jnp.float32)
        for t in range(_NK):
            wp = ws[int(_PAR[t])]
            v = wp
            for _ in range(int(_EXP[t]) - 1):
                v = v * wp
            val = float(_COEF_A[t]) + float(_COEF_B[t]) * v
            hot = ((ri == int(_IDX[t, 0])) & (ci == int(_IDX[t, 1])))
            hotf = hot.astype(jnp.float32)
            logits = logits + val * hotf
            maskf = maskf + hotf
        emat = jnp.exp(logits) * maskf
        inv = 1.0 / jnp.sum(emat, axis=1, keepdims=True)
        return emat * inv


_NCHUNK = 4
_CROWS = 16384


def _body(w_ref, a_hbm, o_hbm, in0, in1, ou0, ou1, *sems):
    ins = [in0, in1]
    ous = [ou0, ou1]
    sin = sems[:_NCHUNK]
    sout = sems[_NCHUNK:]

    def in_cp(c):
        return pltpu.make_async_copy(
            a_hbm.at[pl.ds(c * _CROWS, _CROWS), :], ins[c % 2], sin[c])

    def out_cp(c):
        return pltpu.make_async_copy(
            ous[c % 2], o_hbm.at[pl.ds(c * _CROWS, _CROWS), :], sout[c])

    in_cp(0).start()
    a_mat = _amat(w_ref)
    for c in range(_NCHUNK):
        if c + 1 < _NCHUNK:
            in_cp(c + 1).start()
        in_cp(c).wait()
        if c >= 2:
            out_cp(c - 2).wait()
        ous[c % 2][...] = jnp.dot(ins[c % 2][...], a_mat,
                                  preferred_element_type=jnp.float32)
        out_cp(c).start()
    out_cp(_NCHUNK - 2).wait()
    out_cp(_NCHUNK - 1).wait()


@jax.jit
def kernel(alpha, transition_kernel):
    n = alpha.shape[0]
    w2 = transition_kernel.reshape(1, _N_PARAMS)
    return pl.pallas_call(
        _body,
        in_specs=[
            pl.BlockSpec(memory_space=pltpu.SMEM),
            pl.BlockSpec(memory_space=pl.ANY),
        ],
        out_specs=pl.BlockSpec(memory_space=pl.ANY),
        out_shape=jax.ShapeDtypeStruct((n, _N_STATES), jnp.float32),
        scratch_shapes=(
            [pltpu.VMEM((_CROWS, _N_STATES), jnp.float32)] * 4
            + [pltpu.SemaphoreType.DMA] * (2 * _NCHUNK)
        ),
    )(w2, alpha)


# speedup vs baseline: 1.0329x; 1.0329x over previous
import jax
import jax.numpy as jnp
import numpy as np
from jax.experimental import pallas as pl
from jax.experimental.pallas import tpu as pltpu

_NCODONS = 2
_N_STATES = 24
_N_PARAMS = 10


def _static_structure(nCodons=_NCODONS):
    offset = 8 + 3 * nCodons
    idx = [[0, 0], [0, 1], [1, 2], [2, 3]]
    idx += [[3 + i * 3, 4 + i * 3] for i in range(nCodons)]
    idx += [[4 + i * 3, 5 + i * 3] for i in range(nCodons)]
    idx += [[5 + i * 3, 6 + i * 3] for i in range(nCodons)]
    idx += [[3 + i * 3, offset + i * 3] for i in range(nCodons + 1)]
    idx += [[3 + nCodons * 3, 4 + nCodons * 3]]
    idx += [[offset + i * 3, offset + 1 + i * 3] for i in range(nCodons + 1)]
    idx += [[offset + 1 + i * 3, offset + 2 + i * 3] for i in range(nCodons + 1)]
    idx += [[offset + 2 + i * 3, 4 + i * 3] for i in range(nCodons + 1)]
    idx += [[offset + 2 + i * 3, offset + i * 3] for i in range(nCodons + 1)]
    i_del = [3 + i * 3 for i in range(nCodons) for j in range(nCodons - i)]
    j_del = [4 + j * 3 for i in range(1, nCodons + 1) for j in range(i, nCodons + 1)]
    idx += [[i, j] for i, j in zip(i_del, j_del)]
    idx += [[4 + nCodons * 3, 5 + nCodons * 3]]
    idx += [[5 + nCodons * 3, 6 + nCodons * 3]]
    idx += [[6 + nCodons * 3, 7 + nCodons * 3]]
    idx += [[7 + nCodons * 3, 7 + nCodons * 3]]
    idx += [[7 + nCodons * 3, 8 + nCodons * 3 + (nCodons + 1) * 3]]
    idx += [[8 + nCodons * 3 + (nCodons + 1) * 3,
             8 + nCodons * 3 + (nCodons + 1) * 3]]
    idx = np.array(idx, dtype=np.int32)

    nc = nCodons
    a, b, p, e = [], [], [], []

    def add(ai, bi, pi, ei):
        a.append(ai); b.append(bi); p.append(pi); e.append(ei)

    add(1.0, -1.0, 0, 1)
    add(0.0, 1.0, 0, 1)
    for _ in range(2):
        add(1.0, 0.0, 0, 1)
    k = 1
    for i in range(nc):
        add(0.0, 1.0, k + i, 1)
    k += nc
    for _ in range(2 * nc):
        add(1.0, 0.0, 0, 1)
    for i in range(nc + 1):
        add(0.0, 1.0, k + i, 1)
    k += nc + 1
    add(1.0, -1.0, k - 1, 1)
    for _ in range(2 * (nc + 1)):
        add(1.0, 0.0, 0, 1)
    for i in range(nc + 1):
        add(0.0, 1.0, k + i, 1)
    for i in range(nc + 1):
        add(1.0, -1.0, k + i, 1)
    k += nc + 1
    for i, j in zip(i_del, j_del):
        add(1.0, -1.0, k, 1 + int((j - i) / 3))
    k += 1
    for _ in range(6):
        add(1.0, 0.0, 0, 1)

    assert len(a) == len(idx)
    return (idx, np.asarray(a, np.float32), np.asarray(b, np.float32),
            np.asarray(p, np.int32), np.asarray(e, np.int32))


_IDX, _COEF_A, _COEF_B, _PAR, _EXP = _static_structure()
_NK = len(_IDX)


_W = 384
_NPH = 3

_LIVE_TILES = [(q, p) for q in range(_NPH) for p in range(_NPH)
               if not (q == 0 and p == 2) and not (q == 2 and p == 0)]


def _amat(w_ref):
        ws = [w_ref[0, i] for i in range(_N_PARAMS)]
        ri = jax.lax.broadcasted_iota(jnp.int32, (_N_STATES, _N_STATES), 0)
        ci = jax.lax.broadcasted_iota(jnp.int32, (_N_STATES, _N_STATES), 1)
        logits = jnp.zeros((_N_STATES, _N_STATES), jnp.float32)
        maskf = jnp.zeros((_N_STATES, _N_STATES), jnp.float32)
        for t in range(_NK):
            wp = ws[int(_PAR[t])]
            v = wp
            for _ in range(int(_EXP[t]) - 1):
                v = v * wp
            val = float(_COEF_A[t]) + float(_COEF_B[t]) * v
            hot = ((ri == int(_IDX[t, 0])) & (ci == int(_IDX[t, 1])))
            hotf = hot.astype(jnp.float32)
            logits = logits + val * hotf
            maskf = maskf + hotf
        emat = jnp.exp(logits) * maskf
        inv = 1.0 / jnp.sum(emat, axis=1, keepdims=True)
        return emat * inv


_NCHUNK = 8
_LROWS = 12288
_CL = _LROWS // _NCHUNK


def _kron_apply(x, t_ref):
    b3 = x.shape[0]
    x3 = x.reshape(b3 // _NPH, _NPH, 128)
    xs = [x3[:, q, :] for q in range(_NPH)]
    ys = [None, None, None]
    for q, p in _LIVE_TILES:
        tqp = t_ref[q * 128:(q + 1) * 128, p * 128:(p + 1) * 128]
        contrib = jnp.dot(xs[q], tqp, preferred_element_type=jnp.float32)
        ys[p] = contrib if ys[p] is None else ys[p] + contrib
    return jnp.stack(ys, axis=1).reshape(b3, 128)


def _build_t(w_ref, t_ref):
    a_mat = _amat(w_ref)
    iu = jax.lax.broadcasted_iota(jnp.int32, (_W, _N_STATES), 0)
    ju = jax.lax.broadcasted_iota(jnp.int32, (_W, _N_STATES), 1)
    u = (iu % _N_STATES == ju).astype(jnp.float32)
    jt = jax.lax.broadcasted_iota(jnp.int32, (_N_STATES, _W), 1)
    rt = jax.lax.broadcasted_iota(jnp.int32, (_N_STATES, _W), 0)
    ut = (jt % _N_STATES == rt).astype(jnp.float32)
    a_ut = jnp.dot(a_mat, ut, preferred_element_type=jnp.float32)
    tiled = jnp.dot(u, a_ut, preferred_element_type=jnp.float32)
    bi = jax.lax.broadcasted_iota(jnp.int32, (_W, _W), 0) // _N_STATES
    bj = jax.lax.broadcasted_iota(jnp.int32, (_W, _W), 1) // _N_STATES
    t_ref[...] = jnp.where(bi == bj, tiled, 0.0)


_NIN = 8
_RIN = 8192
_NOB = 4


def _body(w_ref, a_hbm, o_hbm, *refs):
    ins = refs[:_NIN]
    ous = refs[_NIN:_NIN + _NOB]
    a_ref = refs[_NIN + _NOB]
    sin = refs[_NIN + _NOB + 1:_NIN + _NOB + 1 + _NIN]
    sout = refs[_NIN + _NOB + 1 + _NIN:]

    def in_cp(c):
        return pltpu.make_async_copy(
            a_hbm.at[pl.ds(c * _RIN, _RIN), :], ins[c], sin[c])

    def out_cp(c):
        return pltpu.make_async_copy(
            ous[c % _NOB], o_hbm.at[pl.ds(c * _RIN, _RIN), :], sout[c])

    for c in range(_NIN):
        in_cp(c).start()
    a_ref[...] = _amat(w_ref)
    for c in range(_NIN):
        in_cp(c).wait()
        if c >= _NOB:
            out_cp(c - _NOB).wait()
        ous[c % _NOB][...] = jnp.dot(ins[c][...], a_ref[...],
                                     preferred_element_type=jnp.float32)
        out_cp(c).start()
    for c in range(_NIN - _NOB, _NIN):
        out_cp(c).wait()


@jax.jit
def kernel(alpha, transition_kernel):
    n = alpha.shape[0]
    w2 = transition_kernel.reshape(1, _N_PARAMS)
    return pl.pallas_call(
        _body,
        in_specs=[
            pl.BlockSpec(memory_space=pltpu.SMEM),
            pl.BlockSpec(memory_space=pl.ANY),
        ],
        out_specs=pl.BlockSpec(memory_space=pl.ANY),
        out_shape=jax.ShapeDtypeStruct((n, _N_STATES), jnp.float32),
        scratch_shapes=(
            [pltpu.VMEM((_RIN, _N_STATES), jnp.float32)] * (_NIN + _NOB)
            + [pltpu.VMEM((_N_STATES, _N_STATES), jnp.float32)]
            + [pltpu.SemaphoreType.DMA] * (2 * _NIN)
        ),
    )(w2, alpha)
